# Initial kernel scaffold; baseline (speedup 1.0000x reference)
import functools

import jax
import jax.numpy as jnp
from jax import lax
from jax.experimental import pallas as pl
from jax.experimental.pallas import tpu as pltpu
from jax.experimental.pallas import tpu_sc as plsc

NC = 2
NS = 16
NW = NC * NS

C = 128
K = 10
NBUF = 2


@functools.cache
def _build(vocab, dim, n_total):
    per_w = n_total // NW
    nchunk = per_w // C
    t_steps = nchunk // K
    assert per_w * NW == n_total and nchunk * C == per_w
    assert t_steps * K == nchunk and t_steps % NBUF == 0

    mesh = plsc.VectorSubcoreMesh(core_axis_name="c", subcore_axis_name="s")

    @functools.partial(
        pl.kernel,
        out_type=jax.ShapeDtypeStruct((NW, t_steps, K, C, dim), jnp.float32),
        mesh=mesh,
        scratch_types=[
            pltpu.VMEM((nchunk, C), jnp.int32),
            pltpu.VMEM((NBUF, K, C, dim), jnp.float32),
            pltpu.SemaphoreType.DMA,
            pltpu.SemaphoreType.DMA,
            pltpu.SemaphoreType.DMA,
        ],
    )
    def body(idx_hbm, table_hbm, out_hbm, idx_v, rows_v, gat_sem, osem0,
             osem1):
        wid = lax.axis_index("s") * NC + lax.axis_index("c")
        osems = (osem0, osem1)

        pltpu.sync_copy(idx_hbm.at[wid], idx_v)

        def gather_group(t, ph):
            descs = []
            for b in range(K):
                descs.append(
                    pltpu.async_copy(
                        table_hbm.at[idx_v.at[t * K + b]],
                        rows_v.at[ph, b],
                        gat_sem,
                    ))
            for d in descs:
                d.wait()

        def start_out(t, ph):
            pltpu.async_copy(rows_v.at[ph], out_hbm.at[wid, t], osems[ph])

        def wait_out(ph):
            pltpu.make_async_copy(rows_v.at[ph], out_hbm.at[wid, 0],
                                  osems[ph]).wait()

        for ph in range(NBUF):
            gather_group(ph, ph)
            start_out(ph, ph)

        @pl.loop(1, t_steps // NBUF)
        def _(u):
            for ph in range(NBUF):
                t = NBUF * u + ph
                wait_out(ph)
                gather_group(t, ph)
                start_out(t, ph)

        for ph in range(NBUF):
            wait_out(ph)

    return body


def kernel(inputs, weight):
    b, l = inputs.shape
    vocab, dim = weight.shape
    n_total = b * l
    nchunk = n_total // NW // C

    idx3 = inputs.reshape(NW, nchunk, C)
    out = _build(vocab, dim, n_total)(idx3, weight)
    return out.reshape(b, l, dim)


# speedup vs baseline: 1.4918x; 1.4918x over previous
import functools

import jax
import jax.numpy as jnp
from jax import lax
from jax.experimental import pallas as pl
from jax.experimental.pallas import tpu as pltpu
from jax.experimental.pallas import tpu_sc as plsc

NC = 2
NS = 16
NW = NC * NS

C = 128
K = 10
NBUF = 2


@functools.cache
def _build(vocab, dim, n_total):
    per_w = n_total // NW
    nchunk = per_w // C
    t_steps = nchunk // K
    assert per_w * NW == n_total and nchunk * C == per_w
    assert t_steps * K == nchunk and t_steps % NBUF == 0

    mesh = plsc.VectorSubcoreMesh(core_axis_name="c", subcore_axis_name="s")

    @functools.partial(
        pl.kernel,
        out_type=jax.ShapeDtypeStruct((NW, t_steps, K, C, dim), jnp.float32),
        mesh=mesh,
        compiler_params=pltpu.CompilerParams(use_tc_tiling_on_sc=False),
        scratch_types=[
            pltpu.VMEM((nchunk, C), jnp.int32),
            pltpu.VMEM((NBUF, K, C, dim), jnp.float32),
            pltpu.SemaphoreType.DMA,
            pltpu.SemaphoreType.DMA,
            pltpu.SemaphoreType.DMA,
        ],
    )
    def body(idx_hbm, table_hbm, out_hbm, idx_v, rows_v, gat_sem, osem0,
             osem1):
        wid = lax.axis_index("s") * NC + lax.axis_index("c")
        osems = (osem0, osem1)

        pltpu.sync_copy(idx_hbm.at[wid], idx_v)

        def gather_group(t, ph):
            descs = []
            for b in range(K):
                descs.append(
                    pltpu.async_copy(
                        table_hbm.at[idx_v.at[t * K + b]],
                        rows_v.at[ph, b],
                        gat_sem,
                    ))
            for d in descs:
                d.wait()

        def start_out(t, ph):
            pltpu.async_copy(rows_v.at[ph], out_hbm.at[wid, t], osems[ph])

        def wait_out(ph):
            pltpu.make_async_copy(rows_v.at[ph], out_hbm.at[wid, 0],
                                  osems[ph]).wait()

        for ph in range(NBUF):
            gather_group(ph, ph)
            start_out(ph, ph)

        @pl.loop(1, t_steps // NBUF)
        def _(u):
            for ph in range(NBUF):
                t = NBUF * u + ph
                wait_out(ph)
                gather_group(t, ph)
                start_out(t, ph)

        for ph in range(NBUF):
            wait_out(ph)

    return body


def kernel(inputs, weight):
    b, l = inputs.shape
    vocab, dim = weight.shape
    n_total = b * l
    nchunk = n_total // NW // C

    idx3 = inputs.reshape(NW, nchunk, C)
    out = _build(vocab, dim, n_total)(idx3, weight)
    return out.reshape(b, l, dim)


# speedup vs baseline: 1.5005x; 1.0058x over previous
import functools

import jax
import jax.numpy as jnp
from jax import lax
from jax.experimental import pallas as pl
from jax.experimental.pallas import tpu as pltpu
from jax.experimental.pallas import tpu_sc as plsc

NC = 2
NS = 16
NW = NC * NS
G = 1280


@functools.cache
def _build(vocab, dim, n_total):
    per_w = n_total // NW
    t_steps = per_w // G
    assert per_w * NW == n_total and t_steps * G == per_w
    assert t_steps % 2 == 0 and t_steps >= 4

    mesh = plsc.VectorSubcoreMesh(core_axis_name="c", subcore_axis_name="s")

    @functools.partial(
        pl.kernel,
        out_type=jax.ShapeDtypeStruct((NW, t_steps, G, dim), jnp.float32),
        mesh=mesh,
        compiler_params=pltpu.CompilerParams(use_tc_tiling_on_sc=False),
        scratch_types=[
            pltpu.VMEM((t_steps, G), jnp.int32),
            pltpu.VMEM((2, G, dim), jnp.float32),
            pltpu.SemaphoreType.DMA,
            pltpu.SemaphoreType.DMA,
            pltpu.SemaphoreType.DMA,
            pltpu.SemaphoreType.DMA,
        ],
    )
    def body(idx_hbm, table_hbm, out_hbm, idx_v, rows_v, gsem0, gsem1,
             osem0, osem1):
        wid = lax.axis_index("s") * NC + lax.axis_index("c")
        gsems = (gsem0, gsem1)
        osems = (osem0, osem1)

        pltpu.sync_copy(idx_hbm.at[wid], idx_v)

        def start_gather(t, ph):
            pltpu.async_copy(table_hbm.at[idx_v.at[t]], rows_v.at[ph],
                             gsems[ph])

        def wait_gather(ph):
            pltpu.make_async_copy(table_hbm.at[idx_v.at[0]], rows_v.at[ph],
                                  gsems[ph]).wait()

        def start_out(t, ph):
            pltpu.async_copy(rows_v.at[ph], out_hbm.at[wid, t], osems[ph])

        def wait_out(ph):
            pltpu.make_async_copy(rows_v.at[ph], out_hbm.at[wid, 0],
                                  osems[ph]).wait()

        start_gather(0, 0)
        start_gather(1, 1)
        wait_gather(0)
        start_out(0, 0)

        @pl.loop(1, t_steps // 2)
        def _(u):
            for ph in range(2):
                t = 2 * u + ph
                wait_out(ph)
                start_gather(t, ph)
                wait_gather(1 - ph)
                start_out(t - 1, 1 - ph)

        wait_gather(1)
        start_out(t_steps - 1, 1)
        wait_out(0)
        wait_out(1)

    return body


def kernel(inputs, weight):
    b, l = inputs.shape
    vocab, dim = weight.shape
    n_total = b * l
    t_steps = n_total // NW // G

    idx3 = inputs.reshape(NW, t_steps, G)
    out = _build(vocab, dim, n_total)(idx3, weight)
    return out.reshape(b, l, dim)
